# baseline (speedup 1.0000x reference)
import math

import jax
import jax.numpy as jnp
import numpy as np
from jax.experimental import pallas as pl
from jax.experimental.pallas import tpu as pltpu

_SC = sum(math.sin(c + 1) for c in range(5))
_SD = sum(1.0 / (math.cos(d + 1e-9) + 1e-9) for d in range(5))
_SE = sum(math.sqrt(e + 1) for e in range(5))
_SMUL = _SC * _SD * _SE

_CI = np.arange(23)
_CJ = (_CI * 7919) % 23
_CK = (_CJ * 1543) % 23

_POOL = 28 * 28 + 1e-9
_BBLK = 32


def _prep_body(sc_ref, tv_ref, b1_ref, n1_ref, b2_ref, n2_ref, w1_ref, w2_ref):
    def mk(base, noise, ab, ts):
        acc = base * _SMUL
        acc = acc + ab * jnp.sum(acc, axis=1, keepdims=True)
        acc = acc + ts
        r = noise
        for _ in range(3):
            r = r * (r + 1e-7)
        k = acc + r
        return k / (jnp.mean(jnp.abs(k)) + 1e-12)

    ts1 = jnp.sum(tv_ref[0:1, :]) * 1e-12
    ts2 = jnp.sum(tv_ref[1:2, :]) * 1e-12
    w1_ref[...] = mk(b1_ref[...], n1_ref[...], sc_ref[0], ts1)
    w2_ref[...] = mk(b2_ref[...], n2_ref[...], sc_ref[1], ts2)


def _wmix_body(fc_ref, lin_ref, out_ref):
    @pl.when(pl.program_id(1) == 0)
    def _init():
        out_ref[...] = jnp.zeros_like(out_ref)

    out_ref[...] += jnp.dot(
        fc_ref[...], lin_ref[...], preferred_element_type=jnp.float32
    )


def _shift2(x, s, a, b):
    if s != 0:
        x = jnp.concatenate([x[:, s:], x[:, :s]], axis=1)
    ids = jax.lax.broadcasted_iota(jnp.int32, (1, 784), 1)
    xc = ids % 28
    yc = ids // 28
    valid = (xc + b >= 0) & (xc + b < 28) & (yc + a >= 0) & (yc + a < 28)
    return jnp.where(valid, x, 0.0)


def _shift3(x, s, a, b):
    if s != 0:
        x = jnp.concatenate([x[:, :, s:], x[:, :, :s]], axis=2)
    ids = jax.lax.broadcasted_iota(jnp.int32, (1, 1, 784), 2)
    xc = ids % 28
    yc = ids // 28
    valid = (xc + b >= 0) & (xc + b < 28) & (yc + a >= 0) & (yc + a < 28)
    return jnp.where(valid, x, 0.0)


def _conv_body(x_ref, w1_ref, w2_ref, v_ref):
    xb = x_ref[...]
    xs = []
    for dy in range(5):
        for dx in range(5):
            a, b = dy - 2, dx - 2
            xs.append(_shift2(xb, a * 28 + b, a, b))
    xs = jnp.stack(xs, axis=0)
    h1 = jnp.einsum(
        "ok,kbp->obp", w1_ref[...], xs, preferred_element_type=jnp.float32
    )
    h1 = jnp.maximum(h1, 0.0) / _POOL

    hs = []
    for dy in range(5):
        for dx in range(5):
            a, b = dy - 2, dx - 2
            hs.append(_shift3(h1, a * 28 + b, a, b))
    hs = jnp.concatenate(hs, axis=0)
    h2 = jnp.einsum(
        "ok,kbp->obp", w2_ref[...], hs, preferred_element_type=jnp.float32
    )
    h2 = jnp.maximum(h2, 0.0) / _POOL

    ss = jnp.sum(h2 * h2, axis=(0, 2), keepdims=True)
    vn = h2 / (jnp.sqrt(ss) + 1e-20)
    v_ref[...] = vn * (vn + 1e-12)


def _fc_body(v_ref, w_ref, b_ref, y_ref):
    acc = jnp.zeros((256, 16), jnp.float32)
    for o in range(16):
        acc = acc + jax.lax.dot_general(
            v_ref[o],
            w_ref[o],
            (((1,), (0,)), ((), ())),
            preferred_element_type=jnp.float32,
        )
    y = acc + b_ref[...]
    m = jnp.mean(jnp.abs(y[:, :10]))
    y_ref[...] = y / (m + 1e-30)


def kernel(x, base1, a1, b1, thrash1, noise1, base2, a2, b2, thrash2, noise2,
           lin_w, fc_w, fc_b):
    f32 = jnp.float32

    sc = jnp.stack([a1 * b1, a2 * b2])
    tv = jnp.stack([thrash1[_CI, _CJ, _CK], thrash2[_CI, _CJ, _CK]])
    w1n, w2n = pl.pallas_call(
        _prep_body,
        in_specs=[
            pl.BlockSpec(memory_space=pltpu.SMEM),
            pl.BlockSpec(),
            pl.BlockSpec(),
            pl.BlockSpec(),
            pl.BlockSpec(),
            pl.BlockSpec(),
        ],
        out_shape=[
            jax.ShapeDtypeStruct((8, 25), f32),
            jax.ShapeDtypeStruct((128, 25), f32),
        ],
    )(sc, tv, base1.reshape(8, 25), noise1.reshape(8, 25),
      base2.reshape(128, 25), noise2.reshape(128, 25))

    w1L = w1n
    w2L = w2n.reshape(16, 8, 25).transpose(0, 2, 1).reshape(16, 200)

    fcp = jnp.concatenate([fc_w, jnp.zeros((6, 12544), f32)], axis=0)
    blk = 1792
    nb = 12544 // blk
    w2mix = pl.pallas_call(
        _wmix_body,
        grid=(nb, nb),
        in_specs=[
            pl.BlockSpec((16, blk), lambda k, j: (0, j)),
            pl.BlockSpec((blk, blk), lambda k, j: (j, k)),
        ],
        out_specs=pl.BlockSpec((16, blk), lambda k, j: (0, k)),
        out_shape=jax.ShapeDtypeStruct((16, 12544), f32),
        compiler_params=pltpu.CompilerParams(
            dimension_semantics=("arbitrary", "arbitrary"),
        ),
    )(fcp, lin_w)

    v = pl.pallas_call(
        _conv_body,
        grid=(256 // _BBLK,),
        in_specs=[
            pl.BlockSpec((_BBLK, 784), lambda i: (i, 0)),
            pl.BlockSpec((8, 25), lambda i: (0, 0)),
            pl.BlockSpec((16, 200), lambda i: (0, 0)),
        ],
        out_specs=pl.BlockSpec((16, _BBLK, 784), lambda i: (0, i, 0)),
        out_shape=jax.ShapeDtypeStruct((16, 256, 784), f32),
        compiler_params=pltpu.CompilerParams(
            dimension_semantics=("arbitrary",),
        ),
    )(x.reshape(256, 784), w1L, w2L)

    w2r = w2mix.reshape(16, 16, 784).transpose(1, 2, 0)
    fcb = jnp.concatenate([fc_b, jnp.zeros((6,), f32)]).reshape(1, 16)
    y16 = pl.pallas_call(
        _fc_body,
        out_shape=jax.ShapeDtypeStruct((256, 16), f32),
    )(v, w2r, fcb)
    return y16[:, :10]
